# baseline (speedup 1.0000x reference)
import functools

import jax
import jax.numpy as jnp
from jax import lax
from jax.experimental import pallas as pl
from jax.experimental.pallas import tpu as pltpu
from jax.experimental.pallas import tpu_sc as plsc

V = 1000000
D = 64
B = 16384

NC = 2
NS = 16
NW = NC * NS
ROWS_PER_W = B // NW
PAIR = 2
NSTEP = ROWS_PER_W // PAIR
NSLOT = 2

_mesh = plsc.VectorSubcoreMesh(core_axis_name="c", subcore_axis_name="s")


@functools.partial(
    pl.kernel,
    out_type=jax.ShapeDtypeStruct((B,), jnp.float32),
    mesh=_mesh,
    compiler_params=pltpu.CompilerParams(needs_layout_passes=False),
    scratch_types=[
        pltpu.VMEM((ROWS_PER_W,), jnp.int32),
        pltpu.VMEM((ROWS_PER_W,), jnp.int32),
        pltpu.SMEM((ROWS_PER_W,), jnp.float32),
        pltpu.VMEM((NSLOT, D, 128), jnp.float32),
        pltpu.VMEM((NSLOT, D, 128), jnp.float32),
        pltpu.VMEM((ROWS_PER_W,), jnp.float32),
        pltpu.VMEM((ROWS_PER_W,), jnp.float32),
        pltpu.SemaphoreType.DMA,
        pltpu.SemaphoreType.DMA,
        pltpu.SemaphoreType.DMA,
        pltpu.SemaphoreType.DMA,
        pltpu.SemaphoreType.DMA,
    ],
)
def _kmer2vec_sc(y_hbm, lbl_hbm, embt_hbm, wt_hbm, b_hbm, out_hbm,
                 lv, yv, osm, eslab, wslab, brows, out_v,
                 sem_e0, sem_e1, sem_w0, sem_w1, sem_b):
    wid = lax.axis_index("s") * NC + lax.axis_index("c")
    base = wid * ROWS_PER_W

    pltpu.sync_copy(lbl_hbm.at[pl.ds(base, ROWS_PER_W)], lv)
    pltpu.sync_copy(y_hbm.at[pl.ds(base, ROWS_PER_W)], yv)
    cb = pltpu.async_copy(b_hbm.at[lv], brows, sem_b)

    sem_e = (sem_e0, sem_e1)
    sem_w = (sem_w0, sem_w1)
    iota16 = lax.iota(jnp.int32, 16)
    cb.wait()

    def loop_body(t, _):
        rows = t * 16 + iota16
        yvec = plsc.load_gather(yv, [rows])
        lvec = plsc.load_gather(lv, [rows])
        ewin = lax.shift_left(lax.shift_right_logical(yvec, 7), 7)
        wwin = lax.shift_left(lax.shift_right_logical(lvec, 7), 7)

        def fire(i):
            slot = i % NSLOT
            ew = pl.multiple_of(ewin[i], 128)
            pltpu.async_copy(embt_hbm.at[:, pl.ds(ew, 128)],
                             eslab.at[slot], sem_e[slot])
            ww = pl.multiple_of(wwin[i], 128)
            pltpu.async_copy(wt_hbm.at[:, pl.ds(ww, 128)],
                             wslab.at[slot], sem_w[slot])

        def wait_slot(slot):
            pltpu.make_async_copy(embt_hbm.at[:, pl.ds(0, 128)],
                                  eslab.at[slot], sem_e[slot]).wait()
            pltpu.make_async_copy(wt_hbm.at[:, pl.ds(0, 128)],
                                  wslab.at[slot], sem_w[slot]).wait()

        fire(0)
        fire(1)
        for i in range(16):
            slot = i % NSLOT
            wait_slot(slot)
            s16 = jnp.full((16,), slot, jnp.int32)
            lane_e = jnp.full((16,), lax.bitwise_and(yvec[i], 127), jnp.int32)
            lane_w = jnp.full((16,), lax.bitwise_and(lvec[i], 127), jnp.int32)
            p = None
            for k in range(D // 16):
                dk = k * 16 + iota16
                ev = plsc.load_gather(eslab, [s16, dk, lane_e])
                wv = plsc.load_gather(wslab, [s16, dk, lane_w])
                p = ev * wv if p is None else p + ev * wv
            osm[t * 16 + i] = lax.reduce_sum(p, axes=(0,))
            if i + NSLOT < 16:
                fire(i + NSLOT)

        return 0

    lax.fori_loop(0, ROWS_PER_W // 16, loop_body, 0)

    for g in range(ROWS_PER_W // 16):
        vec = jnp.zeros((16,), jnp.float32)
        for j in range(16):
            vec = jnp.where(iota16 == j, osm[g * 16 + j], vec)
        out_v[pl.ds(g * 16, 16)] = vec + brows[pl.ds(g * 16, 16)]

    pltpu.sync_copy(out_v, out_hbm.at[pl.ds(base, ROWS_PER_W)])


def kernel(y, labels, embeddings, nce_weights, nce_biases):
    et = embeddings.T
    wt = nce_weights.T
    yf = y.astype(jnp.int32)
    lf = labels.astype(jnp.int32).reshape(B)
    return _kmer2vec_sc(yf, lf, et, wt, nce_biases)


# speedup vs baseline: 1.1908x; 1.1908x over previous
import functools

import jax
import jax.numpy as jnp
from jax import lax
from jax.experimental import pallas as pl
from jax.experimental.pallas import tpu as pltpu
from jax.experimental.pallas import tpu_sc as plsc

V = 1000000
D = 64
B = 16384

NC = 2
NS = 16
NW = NC * NS
ROWS_PER_W = B // NW
PAIR = 2
NSTEP = ROWS_PER_W // PAIR
NSLOT = 4

_mesh = plsc.VectorSubcoreMesh(core_axis_name="c", subcore_axis_name="s")


@functools.partial(
    pl.kernel,
    out_type=jax.ShapeDtypeStruct((B,), jnp.float32),
    mesh=_mesh,
    compiler_params=pltpu.CompilerParams(needs_layout_passes=False),
    scratch_types=[
        pltpu.VMEM((ROWS_PER_W,), jnp.int32),
        pltpu.VMEM((ROWS_PER_W,), jnp.int32),
        pltpu.SMEM((ROWS_PER_W,), jnp.float32),
        pltpu.VMEM((NSLOT, D, 128), jnp.float32),
        pltpu.VMEM((NSLOT, D, 128), jnp.float32),
        pltpu.VMEM((ROWS_PER_W,), jnp.float32),
        pltpu.VMEM((ROWS_PER_W,), jnp.float32),
    ] + [pltpu.SemaphoreType.DMA] * 9,
)
def _kmer2vec_sc(y_hbm, lbl_hbm, embt_hbm, wt_hbm, b_hbm, out_hbm,
                 lv, yv, osm, eslab, wslab, brows, out_v,
                 sem_e0, sem_e1, sem_e2, sem_e3,
                 sem_w0, sem_w1, sem_w2, sem_w3, sem_b):
    wid = lax.axis_index("s") * NC + lax.axis_index("c")
    base = wid * ROWS_PER_W

    pltpu.sync_copy(lbl_hbm.at[pl.ds(base, ROWS_PER_W)], lv)
    pltpu.sync_copy(y_hbm.at[pl.ds(base, ROWS_PER_W)], yv)
    cb = pltpu.async_copy(b_hbm.at[lv], brows, sem_b)

    sem_e = (sem_e0, sem_e1, sem_e2, sem_e3)
    sem_w = (sem_w0, sem_w1, sem_w2, sem_w3)
    iota16 = lax.iota(jnp.int32, 16)
    cb.wait()

    def loop_body(t, _):
        rows = t * 16 + iota16
        yvec = plsc.load_gather(yv, [rows])
        lvec = plsc.load_gather(lv, [rows])
        ewin = lax.shift_left(lax.shift_right_logical(yvec, 7), 7)
        wwin = lax.shift_left(lax.shift_right_logical(lvec, 7), 7)

        def fire(i):
            slot = i % NSLOT
            ew = pl.multiple_of(ewin[i], 128)
            pltpu.async_copy(embt_hbm.at[:, pl.ds(ew, 128)],
                             eslab.at[slot], sem_e[slot])
            ww = pl.multiple_of(wwin[i], 128)
            pltpu.async_copy(wt_hbm.at[:, pl.ds(ww, 128)],
                             wslab.at[slot], sem_w[slot])

        def wait_slot(slot):
            pltpu.make_async_copy(embt_hbm.at[:, pl.ds(0, 128)],
                                  eslab.at[slot], sem_e[slot]).wait()
            pltpu.make_async_copy(wt_hbm.at[:, pl.ds(0, 128)],
                                  wslab.at[slot], sem_w[slot]).wait()

        for i in range(NSLOT):
            fire(i)
        for i in range(16):
            slot = i % NSLOT
            wait_slot(slot)
            s16 = jnp.full((16,), slot, jnp.int32)
            lane_e = jnp.full((16,), lax.bitwise_and(yvec[i], 127), jnp.int32)
            lane_w = jnp.full((16,), lax.bitwise_and(lvec[i], 127), jnp.int32)
            p = None
            for k in range(D // 16):
                dk = k * 16 + iota16
                ev = plsc.load_gather(eslab, [s16, dk, lane_e])
                wv = plsc.load_gather(wslab, [s16, dk, lane_w])
                p = ev * wv if p is None else p + ev * wv
            osm[t * 16 + i] = lax.reduce_sum(p, axes=(0,))
            if i + NSLOT < 16:
                fire(i + NSLOT)

        return 0

    lax.fori_loop(0, ROWS_PER_W // 16, loop_body, 0)

    for g in range(ROWS_PER_W // 16):
        vec = jnp.zeros((16,), jnp.float32)
        for j in range(16):
            vec = jnp.where(iota16 == j, osm[g * 16 + j], vec)
        out_v[pl.ds(g * 16, 16)] = vec + brows[pl.ds(g * 16, 16)]

    pltpu.sync_copy(out_v, out_hbm.at[pl.ds(base, ROWS_PER_W)])


def kernel(y, labels, embeddings, nce_weights, nce_biases):
    et = embeddings.T
    wt = nce_weights.T
    yf = y.astype(jnp.int32)
    lf = labels.astype(jnp.int32).reshape(B)
    return _kmer2vec_sc(yf, lf, et, wt, nce_biases)


# speedup vs baseline: 1.1996x; 1.0074x over previous
import functools

import jax
import jax.numpy as jnp
from jax import lax
from jax.experimental import pallas as pl
from jax.experimental.pallas import tpu as pltpu
from jax.experimental.pallas import tpu_sc as plsc

V = 1000000
D = 64
B = 16384

NC = 2
NS = 16
NW = NC * NS
ROWS_PER_W = B // NW
PAIR = 2
NSTEP = ROWS_PER_W // PAIR
NSLOT = 6

_mesh = plsc.VectorSubcoreMesh(core_axis_name="c", subcore_axis_name="s")


@functools.partial(
    pl.kernel,
    out_type=jax.ShapeDtypeStruct((B,), jnp.float32),
    mesh=_mesh,
    compiler_params=pltpu.CompilerParams(needs_layout_passes=False),
    scratch_types=[
        pltpu.VMEM((ROWS_PER_W,), jnp.int32),
        pltpu.VMEM((ROWS_PER_W,), jnp.int32),
        pltpu.SMEM((ROWS_PER_W,), jnp.float32),
        pltpu.VMEM((NSLOT, D, 128), jnp.float32),
        pltpu.VMEM((NSLOT, D, 128), jnp.float32),
        pltpu.VMEM((ROWS_PER_W,), jnp.float32),
        pltpu.VMEM((ROWS_PER_W,), jnp.float32),
    ] + [pltpu.SemaphoreType.DMA] * 13,
)
def _kmer2vec_sc(y_hbm, lbl_hbm, embt_hbm, wt_hbm, b_hbm, out_hbm,
                 lv, yv, osm, eslab, wslab, brows, out_v,
                 sem_e0, sem_e1, sem_e2, sem_e3, sem_e4, sem_e5,
                 sem_w0, sem_w1, sem_w2, sem_w3, sem_w4, sem_w5, sem_b):
    wid = lax.axis_index("s") * NC + lax.axis_index("c")
    base = wid * ROWS_PER_W

    pltpu.sync_copy(lbl_hbm.at[pl.ds(base, ROWS_PER_W)], lv)
    pltpu.sync_copy(y_hbm.at[pl.ds(base, ROWS_PER_W)], yv)
    cb = pltpu.async_copy(b_hbm.at[lv], brows, sem_b)

    sem_e = (sem_e0, sem_e1, sem_e2, sem_e3, sem_e4, sem_e5)
    sem_w = (sem_w0, sem_w1, sem_w2, sem_w3, sem_w4, sem_w5)
    iota16 = lax.iota(jnp.int32, 16)
    cb.wait()

    def loop_body(t, _):
        rows = t * 16 + iota16
        yvec = plsc.load_gather(yv, [rows])
        lvec = plsc.load_gather(lv, [rows])
        ewin = lax.shift_left(lax.shift_right_logical(yvec, 7), 7)
        wwin = lax.shift_left(lax.shift_right_logical(lvec, 7), 7)

        def fire(i):
            slot = i % NSLOT
            ew = pl.multiple_of(ewin[i], 128)
            pltpu.async_copy(embt_hbm.at[:, pl.ds(ew, 128)],
                             eslab.at[slot], sem_e[slot])
            ww = pl.multiple_of(wwin[i], 128)
            pltpu.async_copy(wt_hbm.at[:, pl.ds(ww, 128)],
                             wslab.at[slot], sem_w[slot])

        def wait_slot(slot):
            pltpu.make_async_copy(embt_hbm.at[:, pl.ds(0, 128)],
                                  eslab.at[slot], sem_e[slot]).wait()
            pltpu.make_async_copy(wt_hbm.at[:, pl.ds(0, 128)],
                                  wslab.at[slot], sem_w[slot]).wait()

        for i in range(NSLOT):
            fire(i)
        for i in range(16):
            slot = i % NSLOT
            wait_slot(slot)
            s16 = jnp.full((16,), slot, jnp.int32)
            lane_e = jnp.full((16,), lax.bitwise_and(yvec[i], 127), jnp.int32)
            lane_w = jnp.full((16,), lax.bitwise_and(lvec[i], 127), jnp.int32)
            p = None
            for k in range(D // 16):
                dk = k * 16 + iota16
                ev = plsc.load_gather(eslab, [s16, dk, lane_e])
                wv = plsc.load_gather(wslab, [s16, dk, lane_w])
                p = ev * wv if p is None else p + ev * wv
            osm[t * 16 + i] = lax.reduce_sum(p, axes=(0,))
            if i + NSLOT < 16:
                fire(i + NSLOT)

        return 0

    lax.fori_loop(0, ROWS_PER_W // 16, loop_body, 0)

    for g in range(ROWS_PER_W // 16):
        vec = jnp.zeros((16,), jnp.float32)
        for j in range(16):
            vec = jnp.where(iota16 == j, osm[g * 16 + j], vec)
        out_v[pl.ds(g * 16, 16)] = vec + brows[pl.ds(g * 16, 16)]

    pltpu.sync_copy(out_v, out_hbm.at[pl.ds(base, ROWS_PER_W)])


def kernel(y, labels, embeddings, nce_weights, nce_biases):
    et = embeddings.T
    wt = nce_weights.T
    yf = y.astype(jnp.int32)
    lf = labels.astype(jnp.int32).reshape(B)
    return _kmer2vec_sc(yf, lf, et, wt, nce_biases)
